# baseline (speedup 1.0000x reference)
import functools

import jax
import jax.numpy as jnp
from jax import lax
from jax.experimental import pallas as pl
from jax.experimental.pallas import tpu as pltpu
from jax.experimental.pallas import tpu_sc as plsc

N = 10000
E = 160000
D = 64
RD = 8
AD = 9
ED = RD + AD
H = 128
ORB = 9
UW = D + 16

NC = 2
NS = 16
NW = NC * NS
PER_W = E // NW
CH = 1000
RPT = 624
RPT_LAST = N - (NS - 1) * RPT

BE = 3200


def _sc_mesh():
    return plsc.VectorSubcoreMesh(core_axis_name="c", subcore_axis_name="s")


_SC_PARAMS = pltpu.CompilerParams(use_tc_tiling_on_sc=False)


def _sc_gather2(table, idx_il):
    @functools.partial(
        pl.kernel,
        mesh=_sc_mesh(),
        out_type=jax.ShapeDtypeStruct((2 * E, D), jnp.float32),
        scratch_types=[
            pltpu.VMEM((CH,), jnp.int32),
            pltpu.VMEM((CH, D), jnp.float32),
            pltpu.SemaphoreType.DMA,
        ],
        compiler_params=_SC_PARAMS,
    )
    def k(table_h, il_h, out_h, idx1, buf, sem):
        wid = lax.axis_index("c") * NS + lax.axis_index("s")
        base0 = wid * 2 * PER_W

        @pl.loop(0, 2 * PER_W, step=CH)
        def _(off):
            base = base0 + off
            pltpu.sync_copy(il_h.at[pl.ds(base, CH)], idx1)
            pltpu.async_copy(table_h.at[idx1], buf, sem).wait()
            pltpu.sync_copy(buf, out_h.at[pl.ds(base, CH)])

    return k(table, idx_il).reshape(E, 2 * D)


def _sc_scatter(upd, dst, zeros):
    @functools.partial(
        pl.kernel,
        mesh=_sc_mesh(),
        out_type=jax.ShapeDtypeStruct((NC, N, UW), jnp.float32),
        scratch_types=[
            pltpu.VMEM((CH,), jnp.int32),
            pltpu.VMEM((CH, UW), jnp.float32),
            pltpu.VMEM_SHARED((N, UW), jnp.float32),
            pltpu.SemaphoreType.DMA,
        ],
        compiler_params=_SC_PARAMS,
    )
    def k(upd_h, dst_h, z_h, agg_h, idx_v, rows_v, sh_agg, sem):
        c = lax.axis_index("c")
        s = lax.axis_index("s")
        @pl.when(s < NS - 1)
        def _():
            pltpu.sync_copy(z_h.at[pl.ds(s * RPT, RPT)],
                            sh_agg.at[pl.ds(s * RPT, RPT)])

        @pl.when(s == NS - 1)
        def _():
            pltpu.sync_copy(z_h.at[pl.ds((NS - 1) * RPT, RPT_LAST)],
                            sh_agg.at[pl.ds((NS - 1) * RPT, RPT_LAST)])

        plsc.subcore_barrier()

        base0 = (c * NS + s) * PER_W

        @pl.loop(0, PER_W, step=CH)
        def _(off):
            base = base0 + off
            pltpu.sync_copy(dst_h.at[pl.ds(base, CH)], idx_v)
            pltpu.sync_copy(upd_h.at[pl.ds(base, CH), pl.ds(0, UW)], rows_v)
            pltpu.sync_copy(rows_v, sh_agg.at[idx_v], add=True)

        plsc.subcore_barrier()

        @pl.when(s < NS - 1)
        def _():
            pltpu.sync_copy(sh_agg.at[pl.ds(s * RPT, RPT)],
                            agg_h.at[c, pl.ds(s * RPT, RPT)])

        @pl.when(s == NS - 1)
        def _():
            pltpu.sync_copy(sh_agg.at[pl.ds((NS - 1) * RPT, RPT_LAST)],
                            agg_h.at[c, pl.ds((NS - 1) * RPT, RPT_LAST)])

    return k(upd, dst, zeros)


def _silu(v):
    return v * jax.nn.sigmoid(v)


def _lrelu(v):
    return jnp.where(v >= 0, v, 0.01 * v)


def _lin(x, w_ref, b_ref):
    return jnp.dot(x, w_ref[...], preferred_element_type=jnp.float32) + b_ref[...]


def _edge_mlp_body(sfdf_ref, radt_ref, angt_ref,
                   ew0, eb0, ew1, eb1, ew2, eb2, ew3, eb3, ew4, eb4, ew5, eb5,
                   nw0, nb0, nw1, nb1,
                   ef_out, upd_out):
    rad = radt_ref[...].T
    ang = angt_ref[...].T
    ef = jnp.concatenate([rad, ang], axis=1)
    sfdf = sfdf_ref[...]
    df = sfdf[:, D:]
    x = jnp.concatenate([sfdf, ef], axis=1).astype(jnp.bfloat16)
    h = _silu(_lin(x, ew0, eb0)).astype(jnp.bfloat16)
    h = _silu(_lin(h, ew1, eb1)).astype(jnp.bfloat16)
    h = _silu(_lin(h, ew2, eb2)).astype(jnp.bfloat16)
    h = _lrelu(_lin(h, ew3, eb3)).astype(jnp.bfloat16)
    h = _silu(_lin(h, ew4, eb4)).astype(jnp.bfloat16)
    ef_upd = _lin(h, ew5, eb5) + ef
    ef_out[...] = jnp.concatenate(
        [ef_upd, jnp.zeros((ef_upd.shape[0], 32 - ED), jnp.float32)], axis=1)
    msg = jnp.concatenate([df, ef_upd], axis=1).astype(jnp.bfloat16)
    m = _silu(_lin(msg, nw0, nb0)).astype(jnp.bfloat16)
    upd = _lin(m, nw1, nb1)
    upd_out[...] = jnp.concatenate(
        [upd,
         jnp.ones((upd.shape[0], 16), jnp.float32),
         jnp.zeros((upd.shape[0], H - D - 16), jnp.float32)], axis=1)


def _full(shape):
    return pl.BlockSpec(shape, lambda *_: (0,) * len(shape))


def _tc_edge_mlp(sfdf, radt, angt, eu_ws, eu_bs, nu_ws, nu_bs):
    in_specs = [
        pl.BlockSpec((BE, 2 * D), lambda i: (i, 0)),
        pl.BlockSpec((RD, BE), lambda i: (0, i)),
        pl.BlockSpec((AD, BE), lambda i: (0, i)),
    ]
    args = [sfdf, radt, angt]
    for w, b in zip(eu_ws, eu_bs):
        in_specs += [_full(w.shape), _full(b.shape)]
        args += [w, b]
    for w, b in zip(nu_ws, nu_bs):
        in_specs += [_full(w.shape), _full(b.shape)]
        args += [w, b]
    return pl.pallas_call(
        _edge_mlp_body,
        grid=(E // BE,),
        in_specs=in_specs,
        out_specs=[pl.BlockSpec((BE, 32), lambda i: (i, 0)),
                   pl.BlockSpec((BE, H), lambda i: (i, 0))],
        out_shape=[jax.ShapeDtypeStruct((E, 32), jnp.float32),
                   jax.ShapeDtypeStruct((E, H), jnp.float32)],
    )(*args)


def _nodeupd_body(agg_ref, env_ref, out_ref):
    agg = agg_ref[0, :, :D] + agg_ref[1, :, :D]
    deg = agg_ref[0, :, D:D + 1] + agg_ref[1, :, D:D + 1]
    out_ref[...] = agg / jnp.maximum(deg, 1.0) + env_ref[...]


def _tc_nodeupd(agg2, node_env):
    return pl.pallas_call(
        _nodeupd_body,
        in_specs=[_full((NC, N, UW)), _full((N, D))],
        out_specs=pl.BlockSpec((N, D), lambda: (0, 0)),
        out_shape=jax.ShapeDtypeStruct((N, D), jnp.float32),
    )(agg2, node_env)


def _head_body(hsd_ref, efp_ref,
               w0, b0, w1, b1, w2, b2, w3, b3, w4, b4, out_ref):
    efu = efp_ref[...][:, :ED]
    x = jnp.concatenate([hsd_ref[...], efu], axis=1).astype(jnp.bfloat16)
    g = _silu(_lin(x, w0, b0)).astype(jnp.bfloat16)
    g = _silu(_lin(g, w1, b1)).astype(jnp.bfloat16)
    g = _silu(_lin(g, w2, b2)).astype(jnp.bfloat16)
    g = _lrelu(_lin(g, w3, b3)).astype(jnp.bfloat16)
    out_ref[...] = _lin(g, w4, b4).T


def _tc_head(hsd, efp, hd_ws, hd_bs):
    in_specs = [
        pl.BlockSpec((BE, 2 * D), lambda i: (i, 0)),
        pl.BlockSpec((BE, 32), lambda i: (i, 0)),
    ]
    args = [hsd, efp]
    for w, b in zip(hd_ws, hd_bs):
        in_specs += [_full(w.shape), _full(b.shape)]
        args += [w, b]
    return pl.pallas_call(
        _head_body,
        grid=(E // BE,),
        in_specs=in_specs,
        out_specs=pl.BlockSpec((ORB * ORB, BE), lambda i: (0, i)),
        out_shape=jax.ShapeDtypeStruct((ORB * ORB, E), jnp.float32),
    )(*args)


def kernel(node_env, radial, angular, edge_index, node_type,
           nu_w0, nu_b0, nu_w1, nu_b1,
           eu_w0, eu_b0, eu_w1, eu_b1, eu_w2, eu_b2, eu_w3, eu_b3,
           eu_w4, eu_b4, eu_w5, eu_b5,
           hd_w0, hd_b0, hd_w1, hd_b1, hd_w2, hd_b2, hd_w3, hd_b3, hd_w4, hd_b4):
    dst = edge_index[1]
    idx_il = edge_index.T.reshape(2 * E)

    bf = jnp.bfloat16
    eu_ws = [w.astype(bf) for w in (eu_w0, eu_w1, eu_w2, eu_w3, eu_w4, eu_w5)]
    eu_bs = [b.reshape(1, -1) for b in (eu_b0, eu_b1, eu_b2, eu_b3, eu_b4, eu_b5)]
    nu_ws = [w.astype(bf) for w in (nu_w0, nu_w1)]
    nu_bs = [b.reshape(1, -1) for b in (nu_b0, nu_b1)]
    hd_ws = [w.astype(bf) for w in (hd_w0, hd_w1, hd_w2, hd_w3, hd_w4)]
    hd_bs = [b.reshape(1, -1) for b in (hd_b0, hd_b1, hd_b2, hd_b3, hd_b4)]

    sfdf = _sc_gather2(node_env, idx_il)
    efp, upd = _tc_edge_mlp(sfdf, radial.T, angular.T,
                            eu_ws, eu_bs, nu_ws, nu_bs)

    zeros = jnp.zeros((N, UW), jnp.float32)
    agg2 = _sc_scatter(upd, dst, zeros)

    nf = _tc_nodeupd(agg2, node_env)
    hsd = _sc_gather2(nf, idx_il)
    out_t = _tc_head(hsd, efp, hd_ws, hd_bs)
    return out_t.reshape(ORB, ORB, E).transpose(2, 0, 1)


# speedup vs baseline: 1.1312x; 1.1312x over previous
import functools

import jax
import jax.numpy as jnp
from jax import lax
from jax.experimental import pallas as pl
from jax.experimental.pallas import tpu as pltpu
from jax.experimental.pallas import tpu_sc as plsc

N = 10000
E = 160000
D = 64
RD = 8
AD = 9
ED = RD + AD
H = 128
ORB = 9
UW = D + 16

NC = 2
NS = 16
NW = NC * NS
PER_W = E // NW
CH = 1000
RPT = 624
RPT_LAST = N - (NS - 1) * RPT

BE = 3200


def _sc_mesh():
    return plsc.VectorSubcoreMesh(core_axis_name="c", subcore_axis_name="s")


_SC_PARAMS = pltpu.CompilerParams(use_tc_tiling_on_sc=False)


def _sc_gather2(table, src, dst):
    @functools.partial(
        pl.kernel,
        mesh=_sc_mesh(),
        out_type=jax.ShapeDtypeStruct((E, 2 * D), jnp.float32),
        scratch_types=[
            pltpu.VMEM((CH,), jnp.int32),
            pltpu.VMEM((CH,), jnp.int32),
            pltpu.VMEM((CH, D), jnp.float32),
            pltpu.SemaphoreType.DMA,
        ],
        compiler_params=_SC_PARAMS,
    )
    def k(table_h, src_h, dst_h, out_h, idx1, idx2, buf, sem):
        wid = lax.axis_index("c") * NS + lax.axis_index("s")
        base0 = wid * PER_W

        @pl.loop(0, PER_W, step=CH)
        def _(off):
            base = base0 + off
            pltpu.sync_copy(src_h.at[pl.ds(base, CH)], idx1)
            pltpu.sync_copy(dst_h.at[pl.ds(base, CH)], idx2)
            pltpu.async_copy(table_h.at[idx1], buf, sem).wait()
            pltpu.sync_copy(buf, out_h.at[pl.ds(base, CH), pl.ds(0, D)])
            pltpu.async_copy(table_h.at[idx2], buf, sem).wait()
            pltpu.sync_copy(buf, out_h.at[pl.ds(base, CH), pl.ds(D, D)])

    return k(table, src, dst)


def _sc_scatter(upd, dst, zeros):
    @functools.partial(
        pl.kernel,
        mesh=_sc_mesh(),
        out_type=jax.ShapeDtypeStruct((NC, N, UW), jnp.float32),
        scratch_types=[
            pltpu.VMEM((CH,), jnp.int32),
            pltpu.VMEM((CH, UW), jnp.float32),
            pltpu.VMEM_SHARED((N, UW), jnp.float32),
            pltpu.SemaphoreType.DMA,
        ],
        compiler_params=_SC_PARAMS,
    )
    def k(upd_h, dst_h, z_h, agg_h, idx_v, rows_v, sh_agg, sem):
        c = lax.axis_index("c")
        s = lax.axis_index("s")
        @pl.when(s < NS - 1)
        def _():
            pltpu.sync_copy(z_h.at[pl.ds(s * RPT, RPT)],
                            sh_agg.at[pl.ds(s * RPT, RPT)])

        @pl.when(s == NS - 1)
        def _():
            pltpu.sync_copy(z_h.at[pl.ds((NS - 1) * RPT, RPT_LAST)],
                            sh_agg.at[pl.ds((NS - 1) * RPT, RPT_LAST)])

        plsc.subcore_barrier()

        base0 = (c * NS + s) * PER_W

        @pl.loop(0, PER_W, step=CH)
        def _(off):
            base = base0 + off
            pltpu.sync_copy(dst_h.at[pl.ds(base, CH)], idx_v)
            pltpu.sync_copy(upd_h.at[pl.ds(base, CH), pl.ds(0, UW)], rows_v)
            pltpu.sync_copy(rows_v, sh_agg.at[idx_v], add=True)

        plsc.subcore_barrier()

        @pl.when(s < NS - 1)
        def _():
            pltpu.sync_copy(sh_agg.at[pl.ds(s * RPT, RPT)],
                            agg_h.at[c, pl.ds(s * RPT, RPT)])

        @pl.when(s == NS - 1)
        def _():
            pltpu.sync_copy(sh_agg.at[pl.ds((NS - 1) * RPT, RPT_LAST)],
                            agg_h.at[c, pl.ds((NS - 1) * RPT, RPT_LAST)])

    return k(upd, dst, zeros)


def _silu(v):
    return v * jax.nn.sigmoid(v)


def _lrelu(v):
    return jnp.where(v >= 0, v, 0.01 * v)


def _lin(x, w_ref, b_ref):
    return jnp.dot(x, w_ref[...], preferred_element_type=jnp.float32) + b_ref[...]


def _edge_mlp_body(sfdf_ref, radt_ref, angt_ref,
                   ew0, eb0, ew1, eb1, ew2, eb2, ew3, eb3, ew4, eb4, ew5, eb5,
                   nw0, nb0, nw1, nb1,
                   ef_out, upd_out):
    rad = radt_ref[...].T
    ang = angt_ref[...].T
    ef = jnp.concatenate([rad, ang], axis=1)
    sfdf = sfdf_ref[...]
    df = sfdf[:, D:]
    x = jnp.concatenate([sfdf, ef], axis=1).astype(jnp.bfloat16)
    h = _silu(_lin(x, ew0, eb0)).astype(jnp.bfloat16)
    h = _silu(_lin(h, ew1, eb1)).astype(jnp.bfloat16)
    h = _silu(_lin(h, ew2, eb2)).astype(jnp.bfloat16)
    h = _lrelu(_lin(h, ew3, eb3)).astype(jnp.bfloat16)
    h = _silu(_lin(h, ew4, eb4)).astype(jnp.bfloat16)
    ef_upd = _lin(h, ew5, eb5) + ef
    ef_out[...] = jnp.concatenate(
        [ef_upd, jnp.zeros((ef_upd.shape[0], 32 - ED), jnp.float32)], axis=1)
    msg = jnp.concatenate([df, ef_upd], axis=1).astype(jnp.bfloat16)
    m = _silu(_lin(msg, nw0, nb0)).astype(jnp.bfloat16)
    upd = _lin(m, nw1, nb1)
    upd_out[...] = jnp.concatenate(
        [upd,
         jnp.ones((upd.shape[0], 16), jnp.float32),
         jnp.zeros((upd.shape[0], H - D - 16), jnp.float32)], axis=1)


def _full(shape):
    return pl.BlockSpec(shape, lambda *_: (0,) * len(shape))


def _tc_edge_mlp(sfdf, radt, angt, eu_ws, eu_bs, nu_ws, nu_bs):
    in_specs = [
        pl.BlockSpec((BE, 2 * D), lambda i: (i, 0)),
        pl.BlockSpec((RD, BE), lambda i: (0, i)),
        pl.BlockSpec((AD, BE), lambda i: (0, i)),
    ]
    args = [sfdf, radt, angt]
    for w, b in zip(eu_ws, eu_bs):
        in_specs += [_full(w.shape), _full(b.shape)]
        args += [w, b]
    for w, b in zip(nu_ws, nu_bs):
        in_specs += [_full(w.shape), _full(b.shape)]
        args += [w, b]
    return pl.pallas_call(
        _edge_mlp_body,
        grid=(E // BE,),
        in_specs=in_specs,
        out_specs=[pl.BlockSpec((BE, 32), lambda i: (i, 0)),
                   pl.BlockSpec((BE, H), lambda i: (i, 0))],
        out_shape=[jax.ShapeDtypeStruct((E, 32), jnp.float32),
                   jax.ShapeDtypeStruct((E, H), jnp.float32)],
    )(*args)


def _nodeupd_body(agg_ref, env_ref, out_ref):
    agg = agg_ref[0, :, :D] + agg_ref[1, :, :D]
    deg = agg_ref[0, :, D:D + 1] + agg_ref[1, :, D:D + 1]
    out_ref[...] = agg / jnp.maximum(deg, 1.0) + env_ref[...]


def _tc_nodeupd(agg2, node_env):
    return pl.pallas_call(
        _nodeupd_body,
        in_specs=[_full((NC, N, UW)), _full((N, D))],
        out_specs=pl.BlockSpec((N, D), lambda: (0, 0)),
        out_shape=jax.ShapeDtypeStruct((N, D), jnp.float32),
    )(agg2, node_env)


def _head_body(hsd_ref, efp_ref,
               w0, b0, w1, b1, w2, b2, w3, b3, w4, b4, out_ref):
    efu = efp_ref[...][:, :ED]
    x = jnp.concatenate([hsd_ref[...], efu], axis=1).astype(jnp.bfloat16)
    g = _silu(_lin(x, w0, b0)).astype(jnp.bfloat16)
    g = _silu(_lin(g, w1, b1)).astype(jnp.bfloat16)
    g = _silu(_lin(g, w2, b2)).astype(jnp.bfloat16)
    g = _lrelu(_lin(g, w3, b3)).astype(jnp.bfloat16)
    out_ref[...] = _lin(g, w4, b4).T


def _tc_head(hsd, efp, hd_ws, hd_bs):
    in_specs = [
        pl.BlockSpec((BE, 2 * D), lambda i: (i, 0)),
        pl.BlockSpec((BE, 32), lambda i: (i, 0)),
    ]
    args = [hsd, efp]
    for w, b in zip(hd_ws, hd_bs):
        in_specs += [_full(w.shape), _full(b.shape)]
        args += [w, b]
    return pl.pallas_call(
        _head_body,
        grid=(E // BE,),
        in_specs=in_specs,
        out_specs=pl.BlockSpec((ORB * ORB, BE), lambda i: (0, i)),
        out_shape=jax.ShapeDtypeStruct((ORB * ORB, E), jnp.float32),
    )(*args)


def kernel(node_env, radial, angular, edge_index, node_type,
           nu_w0, nu_b0, nu_w1, nu_b1,
           eu_w0, eu_b0, eu_w1, eu_b1, eu_w2, eu_b2, eu_w3, eu_b3,
           eu_w4, eu_b4, eu_w5, eu_b5,
           hd_w0, hd_b0, hd_w1, hd_b1, hd_w2, hd_b2, hd_w3, hd_b3, hd_w4, hd_b4):
    src = edge_index[0]
    dst = edge_index[1]

    bf = jnp.bfloat16
    eu_ws = [w.astype(bf) for w in (eu_w0, eu_w1, eu_w2, eu_w3, eu_w4, eu_w5)]
    eu_bs = [b.reshape(1, -1) for b in (eu_b0, eu_b1, eu_b2, eu_b3, eu_b4, eu_b5)]
    nu_ws = [w.astype(bf) for w in (nu_w0, nu_w1)]
    nu_bs = [b.reshape(1, -1) for b in (nu_b0, nu_b1)]
    hd_ws = [w.astype(bf) for w in (hd_w0, hd_w1, hd_w2, hd_w3, hd_w4)]
    hd_bs = [b.reshape(1, -1) for b in (hd_b0, hd_b1, hd_b2, hd_b3, hd_b4)]

    sfdf = _sc_gather2(node_env, src, dst)
    efp, upd = _tc_edge_mlp(sfdf, radial.T, angular.T,
                            eu_ws, eu_bs, nu_ws, nu_bs)

    zeros = jnp.zeros((N, UW), jnp.float32)
    agg2 = _sc_scatter(upd, dst, zeros)

    nf = _tc_nodeupd(agg2, node_env)
    hsd = _sc_gather2(nf, src, dst)
    out_t = _tc_head(hsd, efp, hd_ws, hd_bs)
    return out_t.reshape(ORB, ORB, E).transpose(2, 0, 1)


# speedup vs baseline: 1.1897x; 1.0517x over previous
import functools

import jax
import jax.numpy as jnp
from jax import lax
from jax.experimental import pallas as pl
from jax.experimental.pallas import tpu as pltpu
from jax.experimental.pallas import tpu_sc as plsc

N = 10000
E = 160000
D = 64
RD = 8
AD = 9
ED = RD + AD
H = 128
ORB = 9
UW = D + 16

NC = 2
NS = 16
NW = NC * NS
PER_W = E // NW
CH = 1000
RPT = 624
RPT_LAST = N - (NS - 1) * RPT

BE = 6400


def _sc_mesh():
    return plsc.VectorSubcoreMesh(core_axis_name="c", subcore_axis_name="s")


_SC_PARAMS = pltpu.CompilerParams(use_tc_tiling_on_sc=False)


def _sc_gather2(table, src, dst):
    @functools.partial(
        pl.kernel,
        mesh=_sc_mesh(),
        out_type=jax.ShapeDtypeStruct((E, 2 * D), jnp.float32),
        scratch_types=[
            pltpu.VMEM((CH,), jnp.int32),
            pltpu.VMEM((CH,), jnp.int32),
            pltpu.VMEM((CH, D), jnp.float32),
            pltpu.SemaphoreType.DMA,
        ],
        compiler_params=_SC_PARAMS,
    )
    def k(table_h, src_h, dst_h, out_h, idx1, idx2, buf, sem):
        wid = lax.axis_index("c") * NS + lax.axis_index("s")
        base0 = wid * PER_W

        @pl.loop(0, PER_W, step=CH)
        def _(off):
            base = base0 + off
            pltpu.sync_copy(src_h.at[pl.ds(base, CH)], idx1)
            pltpu.sync_copy(dst_h.at[pl.ds(base, CH)], idx2)
            pltpu.async_copy(table_h.at[idx1], buf, sem).wait()
            pltpu.sync_copy(buf, out_h.at[pl.ds(base, CH), pl.ds(0, D)])
            pltpu.async_copy(table_h.at[idx2], buf, sem).wait()
            pltpu.sync_copy(buf, out_h.at[pl.ds(base, CH), pl.ds(D, D)])

    return k(table, src, dst)


def _sc_scatter(upd, dst, zeros):
    @functools.partial(
        pl.kernel,
        mesh=_sc_mesh(),
        out_type=jax.ShapeDtypeStruct((NC, N, UW), jnp.float32),
        scratch_types=[
            pltpu.VMEM((CH,), jnp.int32),
            pltpu.VMEM((CH, UW), jnp.float32),
            pltpu.VMEM_SHARED((N, UW), jnp.float32),
            pltpu.SemaphoreType.DMA,
        ],
        compiler_params=_SC_PARAMS,
    )
    def k(upd_h, dst_h, z_h, agg_h, idx_v, rows_v, sh_agg, sem):
        c = lax.axis_index("c")
        s = lax.axis_index("s")
        @pl.when(s < NS - 1)
        def _():
            pltpu.sync_copy(z_h.at[pl.ds(s * RPT, RPT)],
                            sh_agg.at[pl.ds(s * RPT, RPT)])

        @pl.when(s == NS - 1)
        def _():
            pltpu.sync_copy(z_h.at[pl.ds((NS - 1) * RPT, RPT_LAST)],
                            sh_agg.at[pl.ds((NS - 1) * RPT, RPT_LAST)])

        plsc.subcore_barrier()

        base0 = (c * NS + s) * PER_W

        @pl.loop(0, PER_W, step=CH)
        def _(off):
            base = base0 + off
            pltpu.sync_copy(dst_h.at[pl.ds(base, CH)], idx_v)
            pltpu.sync_copy(upd_h.at[pl.ds(base, CH), pl.ds(0, UW)], rows_v)
            pltpu.sync_copy(rows_v, sh_agg.at[idx_v], add=True)

        plsc.subcore_barrier()

        @pl.when(s < NS - 1)
        def _():
            pltpu.sync_copy(sh_agg.at[pl.ds(s * RPT, RPT)],
                            agg_h.at[c, pl.ds(s * RPT, RPT)])

        @pl.when(s == NS - 1)
        def _():
            pltpu.sync_copy(sh_agg.at[pl.ds((NS - 1) * RPT, RPT_LAST)],
                            agg_h.at[c, pl.ds((NS - 1) * RPT, RPT_LAST)])

    return k(upd, dst, zeros)


def _silu(v):
    return v * jax.nn.sigmoid(v)


def _lrelu(v):
    return jnp.where(v >= 0, v, 0.01 * v)


def _lin(x, w_ref, b_ref):
    return jnp.dot(x, w_ref[...], preferred_element_type=jnp.float32) + b_ref[...]


def _edge_mlp_body(sfdf_ref, radt_ref, angt_ref,
                   ew0, eb0, ew1, eb1, ew2, eb2, ew3, eb3, ew4, eb4, ew5, eb5,
                   nw0, nb0, nw1, nb1,
                   ef_out, upd_out):
    rad = radt_ref[...].T
    ang = angt_ref[...].T
    ef = jnp.concatenate([rad, ang], axis=1)
    sfdf = sfdf_ref[...]
    df = sfdf[:, D:]
    x = jnp.concatenate([sfdf, ef], axis=1).astype(jnp.bfloat16)
    h = _silu(_lin(x, ew0, eb0)).astype(jnp.bfloat16)
    h = _silu(_lin(h, ew1, eb1)).astype(jnp.bfloat16)
    h = _silu(_lin(h, ew2, eb2)).astype(jnp.bfloat16)
    h = _lrelu(_lin(h, ew3, eb3)).astype(jnp.bfloat16)
    h = _silu(_lin(h, ew4, eb4)).astype(jnp.bfloat16)
    ef_upd = _lin(h, ew5, eb5) + ef
    ef_out[...] = jnp.concatenate(
        [ef_upd, jnp.zeros((ef_upd.shape[0], 32 - ED), jnp.float32)], axis=1)
    msg = jnp.concatenate([df, ef_upd], axis=1).astype(jnp.bfloat16)
    m = _silu(_lin(msg, nw0, nb0)).astype(jnp.bfloat16)
    upd = _lin(m, nw1, nb1)
    upd_out[...] = jnp.concatenate(
        [upd,
         jnp.ones((upd.shape[0], 16), jnp.float32),
         jnp.zeros((upd.shape[0], H - D - 16), jnp.float32)], axis=1)


def _full(shape):
    return pl.BlockSpec(shape, lambda *_: (0,) * len(shape))


def _tc_edge_mlp(sfdf, radt, angt, eu_ws, eu_bs, nu_ws, nu_bs):
    in_specs = [
        pl.BlockSpec((BE, 2 * D), lambda i: (i, 0)),
        pl.BlockSpec((RD, BE), lambda i: (0, i)),
        pl.BlockSpec((AD, BE), lambda i: (0, i)),
    ]
    args = [sfdf, radt, angt]
    for w, b in zip(eu_ws, eu_bs):
        in_specs += [_full(w.shape), _full(b.shape)]
        args += [w, b]
    for w, b in zip(nu_ws, nu_bs):
        in_specs += [_full(w.shape), _full(b.shape)]
        args += [w, b]
    return pl.pallas_call(
        _edge_mlp_body,
        grid=(E // BE,),
        in_specs=in_specs,
        out_specs=[pl.BlockSpec((BE, 32), lambda i: (i, 0)),
                   pl.BlockSpec((BE, H), lambda i: (i, 0))],
        out_shape=[jax.ShapeDtypeStruct((E, 32), jnp.float32),
                   jax.ShapeDtypeStruct((E, H), jnp.float32)],
    )(*args)


def _nodeupd_body(agg_ref, env_ref, out_ref):
    agg = agg_ref[0, :, :D] + agg_ref[1, :, :D]
    deg = agg_ref[0, :, D:D + 1] + agg_ref[1, :, D:D + 1]
    out_ref[...] = agg / jnp.maximum(deg, 1.0) + env_ref[...]


def _tc_nodeupd(agg2, node_env):
    return pl.pallas_call(
        _nodeupd_body,
        in_specs=[_full((NC, N, UW)), _full((N, D))],
        out_specs=pl.BlockSpec((N, D), lambda: (0, 0)),
        out_shape=jax.ShapeDtypeStruct((N, D), jnp.float32),
    )(agg2, node_env)


def _head_body(hsd_ref, efp_ref,
               w0, b0, w1, b1, w2, b2, w3, b3, w4, b4, out_ref):
    efu = efp_ref[...][:, :ED]
    x = jnp.concatenate([hsd_ref[...], efu], axis=1).astype(jnp.bfloat16)
    g = _silu(_lin(x, w0, b0)).astype(jnp.bfloat16)
    g = _silu(_lin(g, w1, b1)).astype(jnp.bfloat16)
    g = _silu(_lin(g, w2, b2)).astype(jnp.bfloat16)
    g = _lrelu(_lin(g, w3, b3)).astype(jnp.bfloat16)
    out_ref[...] = _lin(g, w4, b4).T


def _tc_head(hsd, efp, hd_ws, hd_bs):
    in_specs = [
        pl.BlockSpec((BE, 2 * D), lambda i: (i, 0)),
        pl.BlockSpec((BE, 32), lambda i: (i, 0)),
    ]
    args = [hsd, efp]
    for w, b in zip(hd_ws, hd_bs):
        in_specs += [_full(w.shape), _full(b.shape)]
        args += [w, b]
    return pl.pallas_call(
        _head_body,
        grid=(E // BE,),
        in_specs=in_specs,
        out_specs=pl.BlockSpec((ORB * ORB, BE), lambda i: (0, i)),
        out_shape=jax.ShapeDtypeStruct((ORB * ORB, E), jnp.float32),
    )(*args)


def kernel(node_env, radial, angular, edge_index, node_type,
           nu_w0, nu_b0, nu_w1, nu_b1,
           eu_w0, eu_b0, eu_w1, eu_b1, eu_w2, eu_b2, eu_w3, eu_b3,
           eu_w4, eu_b4, eu_w5, eu_b5,
           hd_w0, hd_b0, hd_w1, hd_b1, hd_w2, hd_b2, hd_w3, hd_b3, hd_w4, hd_b4):
    src = edge_index[0]
    dst = edge_index[1]

    bf = jnp.bfloat16
    eu_ws = [w.astype(bf) for w in (eu_w0, eu_w1, eu_w2, eu_w3, eu_w4, eu_w5)]
    eu_bs = [b.reshape(1, -1) for b in (eu_b0, eu_b1, eu_b2, eu_b3, eu_b4, eu_b5)]
    nu_ws = [w.astype(bf) for w in (nu_w0, nu_w1)]
    nu_bs = [b.reshape(1, -1) for b in (nu_b0, nu_b1)]
    hd_ws = [w.astype(bf) for w in (hd_w0, hd_w1, hd_w2, hd_w3, hd_w4)]
    hd_bs = [b.reshape(1, -1) for b in (hd_b0, hd_b1, hd_b2, hd_b3, hd_b4)]

    sfdf = _sc_gather2(node_env, src, dst)
    efp, upd = _tc_edge_mlp(sfdf, radial.T, angular.T,
                            eu_ws, eu_bs, nu_ws, nu_bs)

    zeros = jnp.zeros((N, UW), jnp.float32)
    agg2 = _sc_scatter(upd, dst, zeros)

    nf = _tc_nodeupd(agg2, node_env)
    hsd = _sc_gather2(nf, src, dst)
    out_t = _tc_head(hsd, efp, hd_ws, hd_bs)
    return out_t.reshape(ORB, ORB, E).transpose(2, 0, 1)
